# baseline (speedup 1.0000x reference)
import functools

import jax
import jax.numpy as jnp
from jax import lax
from jax.experimental import pallas as pl
from jax.experimental.pallas import tpu as pltpu
from jax.experimental.pallas import tpu_sc as plsc

TAU = 0.8
ALPHA = 0.3
NC, NS, L = 2, 16, 16
NW = NC * NS
G = 8


@functools.lru_cache(maxsize=None)
def _build_sc_loss(bt: int, v: int):
    tpw = bt // NW
    ng = tpw // L
    mesh = plsc.VectorSubcoreMesh(
        core_axis_name="c", subcore_axis_name="s",
        num_cores=NC, num_subcores=NS)

    @functools.partial(
        pl.kernel,
        out_type=(
            jax.ShapeDtypeStruct((NW, 2, L), jnp.float32),
            jax.ShapeDtypeStruct((bt, L), jnp.float32),
            jax.ShapeDtypeStruct((bt, L), jnp.float32),
        ),
        mesh=mesh,
        scratch_types=[
            pltpu.VMEM((tpw,), jnp.int32),
            pltpu.VMEM((tpw,), jnp.float32),
            pltpu.VMEM((L, v), jnp.float32),
            pltpu.VMEM((G * v,), jnp.float32),
            pltpu.VMEM((L,), jnp.float32),
            pltpu.VMEM((L, L), jnp.float32),
            pltpu.VMEM((L, L), jnp.float32),
            pltpu.VMEM((2, L), jnp.float32),
            pltpu.SemaphoreType.DMA,
            pltpu.SemaphoreType.DMA,
            pltpu.SemaphoreType.DMA,
        ],
    )
    def sc_loss(x_hbm, tgt_hbm, msk_hbm, dist_hbm,
                out_hbm, esum_hbm, dot_hbm,
                tgt_v, msk_v, rows_e, rows_x, gt_v, st_e, st_p, st_g,
                sem_e, sem_x, sem_g):
        wid = lax.axis_index("c") * NS + lax.axis_index("s")
        base = wid * tpw
        inv_tau = 1.0 / TAU
        pltpu.sync_copy(tgt_hbm.at[pl.ds(base, tpw)], tgt_v)
        pltpu.sync_copy(msk_hbm.at[pl.ds(base, tpw)], msk_v)
        zero = jnp.zeros((L,), jnp.float32)
        lane = lax.broadcasted_iota(jnp.int32, (L,), 0)

        def group_body(g, carry):
            acc_g, acc_m = carry
            idx16 = tgt_v[pl.ds(g * L, L)]
            m16 = msk_v[pl.ds(g * L, L)]
            c_e = pltpu.async_copy(dist_hbm.at[idx16], rows_e, sem_e)
            offs = (base + g * L + lane) * v + idx16
            c_g = pltpu.async_copy(x_hbm.at[offs], gt_v, sem_g)
            c_g.wait()
            acc_g = acc_g + gt_v[...] * m16
            acc_m = acc_m + m16
            c_e.wait()
            for h in range(L // G):
                c_x = pltpu.async_copy(
                    x_hbm.at[pl.ds((base + g * L + h * G) * v, G * v)],
                    rows_x, sem_x)
                c_x.wait()

                def row_body(r, _):
                    def col_body(i, c3):
                        v_e, v_p = c3
                        d = rows_e[h * G + r, pl.ds(i * L, L)]
                        x = rows_x[pl.ds(r * v + i * L, L)]
                        e = jnp.exp(d * inv_tau - inv_tau)
                        return (v_e + e, v_p + x * e)

                    v_e, v_p = lax.fori_loop(0, v // L, col_body, (zero, zero))
                    st_e[h * G + r, :] = v_e
                    st_p[h * G + r, :] = v_p
                    return 0

                lax.fori_loop(0, G, row_body, 0)
            pltpu.sync_copy(st_e, esum_hbm.at[pl.ds(base + g * L, L)])
            pltpu.sync_copy(st_p, dot_hbm.at[pl.ds(base + g * L, L)])
            return (acc_g, acc_m)

        acc_g, acc_m = lax.fori_loop(0, ng, group_body, (zero, zero))
        st_g[0, :] = acc_g
        st_g[1, :] = acc_m
        pltpu.sync_copy(st_g, out_hbm.at[wid])

    return sc_loss


def kernel(input, target, mask, pre_scores, Dist):
    b, t, v = input.shape
    bt = b * t
    x = input.reshape(bt * v)
    tgt = target.reshape(bt).astype(jnp.int32)
    msk = mask.reshape(bt)
    parts, esumv, dotv = _build_sc_loss(bt, v)(x, tgt, msk, Dist)
    s = jnp.sum(parts, axis=(0, 2))
    s_e = jnp.vdot(msk, jnp.sum(esumv, axis=1))
    s_p = jnp.vdot(msk, jnp.sum(dotv, axis=1))
    real = -s[0] / s[1]
    smooth = -s_p / s_e
    return (real, ALPHA * smooth + (1.0 - ALPHA) * real)


# speedup vs baseline: 1.7877x; 1.7877x over previous
import functools

import jax
import jax.numpy as jnp
from jax import lax
from jax.experimental import pallas as pl
from jax.experimental.pallas import tpu as pltpu
from jax.experimental.pallas import tpu_sc as plsc

TAU = 0.8
ALPHA = 0.3
NC, NS, L = 2, 16, 16
NW = NC * NS
GP = 4
UNROLL = 8


@functools.lru_cache(maxsize=None)
def _build_sc_loss(bt: int, v: int):
    tpw = bt // NW
    ng16 = tpw // L
    ngp = tpw // GP
    pairs = ngp // 2
    mesh = plsc.VectorSubcoreMesh(
        core_axis_name="c", subcore_axis_name="s",
        num_cores=NC, num_subcores=NS)

    @functools.partial(
        pl.kernel,
        out_type=(
            jax.ShapeDtypeStruct((NW, 2, L), jnp.float32),
            jax.ShapeDtypeStruct((bt, L), jnp.float32),
            jax.ShapeDtypeStruct((bt, L), jnp.float32),
        ),
        mesh=mesh,
        scratch_types=[
            pltpu.VMEM((tpw,), jnp.int32),
            pltpu.VMEM((ngp, GP), jnp.int32),
            pltpu.VMEM((tpw,), jnp.float32),
            pltpu.VMEM((GP, v), jnp.float32),
            pltpu.VMEM((GP, v), jnp.float32),
            pltpu.VMEM((GP * v,), jnp.float32),
            pltpu.VMEM((GP * v,), jnp.float32),
            pltpu.VMEM((tpw,), jnp.float32),
            pltpu.VMEM((tpw, L), jnp.float32),
            pltpu.VMEM((tpw, L), jnp.float32),
            pltpu.VMEM((2, L), jnp.float32),
            pltpu.SemaphoreType.DMA,
            pltpu.SemaphoreType.DMA,
            pltpu.SemaphoreType.DMA,
            pltpu.SemaphoreType.DMA,
            pltpu.SemaphoreType.DMA,
        ],
    )
    def sc_loss(x_hbm, tgt_hbm, tgt2_hbm, msk_hbm, dist_hbm,
                out_hbm, esum_hbm, dot_hbm,
                tgt_v, tgt2_v, msk_v, e0, e1, x0, x1, gt_v, es_all, dt_all,
                st_g, sem_e0, sem_e1, sem_x0, sem_x1, sem_g):
        wid = lax.axis_index("c") * NS + lax.axis_index("s")
        base = wid * tpw
        inv_tau = 1.0 / TAU
        pltpu.sync_copy(tgt_hbm.at[pl.ds(base, tpw)], tgt_v)
        pltpu.sync_copy(
            tgt2_hbm.at[pl.ds(pl.multiple_of(wid * ngp, 8), ngp)], tgt2_v)
        pltpu.sync_copy(msk_hbm.at[pl.ds(base, tpw)], msk_v)
        zero = jnp.zeros((L,), jnp.float32)
        lane = lax.broadcasted_iota(jnp.int32, (L,), 0)

        gt_copies = []
        for g16 in range(ng16):
            idx16 = tgt_v[pl.ds(g16 * L, L)]
            offs = (base + g16 * L + lane) * v + idx16
            gt_copies.append(pltpu.async_copy(
                x_hbm.at[offs], gt_v.at[pl.ds(g16 * L, L)], sem_g))

        def issue(g, ebuf, xbuf, sem_e, sem_x):
            pltpu.async_copy(dist_hbm.at[tgt2_v.at[g]], ebuf, sem_e)
            pltpu.async_copy(
                x_hbm.at[pl.ds((base + g * GP) * v, GP * v)], xbuf, sem_x)

        def drain(ebuf, xbuf, sem_e, sem_x):
            pltpu.make_async_copy(dist_hbm.at[tgt2_v.at[0]],
                                  ebuf, sem_e).wait()
            pltpu.make_async_copy(x_hbm.at[pl.ds(0, GP * v)],
                                  xbuf, sem_x).wait()

        def compute(g, ebuf, xbuf):
            for r in range(GP):
                tok = g * GP + r

                def col_body(i, c3):
                    v_e, v_p = c3
                    for u in range(UNROLL):
                        d = ebuf[r, pl.ds((i * UNROLL + u) * L, L)]
                        x = xbuf[pl.ds(r * v + (i * UNROLL + u) * L, L)]
                        e = jnp.exp(d * inv_tau - inv_tau)
                        v_e = v_e + e
                        v_p = v_p + x * e
                    return (v_e, v_p)

                v_e, v_p = lax.fori_loop(
                    0, v // (L * UNROLL), col_body, (zero, zero))
                es_all[tok, :] = v_e
                dt_all[tok, :] = v_p

        issue(0, e0, x0, sem_e0, sem_x0)

        def pair_body(k, _):
            g0 = 2 * k
            drain(e0, x0, sem_e0, sem_x0)
            issue(g0 + 1, e1, x1, sem_e1, sem_x1)
            compute(g0, e0, x0)
            drain(e1, x1, sem_e1, sem_x1)
            issue(jnp.minimum(g0 + 2, ngp - 1), e0, x0, sem_e0, sem_x0)
            compute(g0 + 1, e1, x1)
            return 0

        lax.fori_loop(0, pairs, pair_body, 0)
        drain(e0, x0, sem_e0, sem_x0)

        for c in gt_copies:
            c.wait()
        acc_g = zero
        acc_m = zero
        for g16 in range(ng16):
            m16 = msk_v[pl.ds(g16 * L, L)]
            acc_g = acc_g + gt_v[pl.ds(g16 * L, L)] * m16
            acc_m = acc_m + m16
        st_g[0, :] = acc_g
        st_g[1, :] = acc_m
        pltpu.sync_copy(st_g, out_hbm.at[wid])
        pltpu.sync_copy(es_all, esum_hbm.at[pl.ds(base, tpw)])
        pltpu.sync_copy(dt_all, dot_hbm.at[pl.ds(base, tpw)])

    return sc_loss


def kernel(input, target, mask, pre_scores, Dist):
    b, t, v = input.shape
    bt = b * t
    x = input.reshape(bt * v)
    tgt = target.reshape(bt).astype(jnp.int32)
    tgt2 = tgt.reshape(bt // GP, GP)
    msk = mask.reshape(bt)
    parts, esumv, dotv = _build_sc_loss(bt, v)(x, tgt, tgt2, msk, Dist)
    s = jnp.sum(parts, axis=(0, 2))
    s_e = jnp.vdot(msk, jnp.sum(esumv, axis=1))
    s_p = jnp.vdot(msk, jnp.sum(dotv, axis=1))
    real = -s[0] / s[1]
    smooth = -s_p / s_e
    return (real, ALPHA * smooth + (1.0 - ALPHA) * real)
